# Initial kernel scaffold; baseline (speedup 1.0000x reference)
import jax
import jax.numpy as jnp
from jax.experimental import pallas as pl


def kernel(x, pos_table):
    raise NotImplementedError("write your pallas kernel here")


# speedup vs baseline: 1.4891x; 1.4891x over previous
import jax
import jax.numpy as jnp
from jax.experimental import pallas as pl
from jax.experimental.pallas import tpu as pltpu

BLOCK_ROWS = 512


def _add_kernel(x_ref, pos_ref, out_ref):
    out_ref[...] = x_ref[...] + pos_ref[...]


def kernel(x, pos_table):
    batch, ctx, dim = x.shape
    n_blocks = ctx // BLOCK_ROWS
    return pl.pallas_call(
        _add_kernel,
        grid=(n_blocks, batch),
        in_specs=[
            pl.BlockSpec((1, BLOCK_ROWS, dim), lambda i, j: (j, i, 0)),
            pl.BlockSpec((BLOCK_ROWS, dim), lambda i, j: (i, 0)),
        ],
        out_specs=pl.BlockSpec((1, BLOCK_ROWS, dim), lambda i, j: (j, i, 0)),
        out_shape=jax.ShapeDtypeStruct(x.shape, x.dtype),
        compiler_params=pltpu.CompilerParams(
            dimension_semantics=("arbitrary", "arbitrary"),
        ),
    )(x, pos_table)


# speedup vs baseline: 1.6658x; 1.1187x over previous
import jax
import jax.numpy as jnp
from jax.experimental import pallas as pl
from jax.experimental.pallas import tpu as pltpu

BLOCK_ROWS = 1024


def _add_kernel(x_ref, pos_ref, out_ref):
    out_ref[...] = x_ref[...] + pos_ref[...]


def kernel(x, pos_table):
    batch, ctx, dim = x.shape
    n_blocks = ctx // BLOCK_ROWS
    return pl.pallas_call(
        _add_kernel,
        grid=(n_blocks, batch),
        in_specs=[
            pl.BlockSpec((1, BLOCK_ROWS, dim), lambda i, j: (j, i, 0)),
            pl.BlockSpec((BLOCK_ROWS, dim), lambda i, j: (i, 0)),
        ],
        out_specs=pl.BlockSpec((1, BLOCK_ROWS, dim), lambda i, j: (j, i, 0)),
        out_shape=jax.ShapeDtypeStruct(x.shape, x.dtype),
        compiler_params=pltpu.CompilerParams(
            dimension_semantics=("arbitrary", "arbitrary"),
        ),
    )(x, pos_table)


# speedup vs baseline: 1.7326x; 1.0401x over previous
import jax
import jax.numpy as jnp
from jax.experimental import pallas as pl
from jax.experimental.pallas import tpu as pltpu

BLOCK_ROWS = 2048


def _add_kernel(x_ref, pos_ref, out_ref):
    out_ref[...] = x_ref[...] + pos_ref[...]


def kernel(x, pos_table):
    batch, ctx, dim = x.shape
    n_blocks = ctx // BLOCK_ROWS
    return pl.pallas_call(
        _add_kernel,
        grid=(n_blocks, batch),
        in_specs=[
            pl.BlockSpec((1, BLOCK_ROWS, dim), lambda i, j: (j, i, 0)),
            pl.BlockSpec((BLOCK_ROWS, dim), lambda i, j: (i, 0)),
        ],
        out_specs=pl.BlockSpec((1, BLOCK_ROWS, dim), lambda i, j: (j, i, 0)),
        out_shape=jax.ShapeDtypeStruct(x.shape, x.dtype),
        compiler_params=pltpu.CompilerParams(
            dimension_semantics=("arbitrary", "arbitrary"),
        ),
    )(x, pos_table)


# speedup vs baseline: 1.7380x; 1.0031x over previous
import jax
import jax.numpy as jnp
from jax.experimental import pallas as pl
from jax.experimental.pallas import tpu as pltpu

BLOCK_ROWS = 2048


def _add_kernel(x_ref, pos_ref, out_ref):
    out_ref[...] = x_ref[...] + pos_ref[...]


def kernel(x, pos_table):
    batch, ctx, dim = x.shape
    n_blocks = ctx // BLOCK_ROWS
    return pl.pallas_call(
        _add_kernel,
        grid=(n_blocks, batch),
        in_specs=[
            pl.BlockSpec((1, BLOCK_ROWS, dim), lambda i, j: (j, i, 0)),
            pl.BlockSpec((BLOCK_ROWS, dim), lambda i, j: (i, 0)),
        ],
        out_specs=pl.BlockSpec((1, BLOCK_ROWS, dim), lambda i, j: (j, i, 0)),
        out_shape=jax.ShapeDtypeStruct(x.shape, x.dtype),
        compiler_params=pltpu.CompilerParams(
            dimension_semantics=("parallel", "arbitrary"),
        ),
    )(x, pos_table)
